# baseline (speedup 1.0000x reference)
import functools

import jax
import jax.numpy as jnp
from jax.experimental import pallas as pl
from jax.experimental.pallas import tpu as pltpu

_ALPHA = 0.2
_PREC = jax.lax.Precision.DEFAULT


def _soft_assign(z, mu):
    zsq = jnp.sum(z * z, axis=1, keepdims=True)
    musq = jnp.sum(mu * mu, axis=1)
    cross = jax.lax.dot_general(
        z, mu, dimension_numbers=(((1,), (1,)), ((), ())),
        preferred_element_type=jnp.float32, precision=_PREC)
    d2 = zsq + musq[None, :] - 2.0 * cross
    q = 1.0 / (1.0 + d2 / _ALPHA + 1e-8)
    q = q ** (_ALPHA + 1.0)
    return q / jnp.sum(q, axis=1, keepdims=True)


def _main_kernel(adj_ref, x_hbm, wbm_hbm, z_ref, q_ref,
                 xv, wbmv, sup_ref, sem, *, bm, nfeat, nclust):
    i = pl.program_id(0)
    ni = pl.num_programs(0)

    @pl.when(i == 0)
    def _():
        cx = pltpu.make_async_copy(x_hbm, xv, sem)
        cx.start()
        cw = pltpu.make_async_copy(wbm_hbm, wbmv, sem)
        cw.start()
        cx.wait()
        cw.wait()
        sup = jnp.dot(xv[...], wbmv[:nfeat, :],
                      preferred_element_type=jnp.float32,
                      precision=_PREC)
        sup_ref[...] = sup.astype(jnp.bfloat16)

    mu = wbmv[nfeat + 1:nfeat + 1 + nclust, :]

    @pl.when(i > 0)
    def _():
        zp = z_ref[pl.ds((i - 1) * bm, bm), :]
        q_ref[pl.ds((i - 1) * bm, bm), :] = _soft_assign(zp, mu)

    z = jnp.dot(adj_ref[...].astype(jnp.bfloat16), sup_ref[...],
                preferred_element_type=jnp.float32,
                precision=_PREC) + wbmv[nfeat:nfeat + 1, :]
    z_ref[pl.ds(i * bm, bm), :] = z

    @pl.when(i == ni - 1)
    def _():
        q_ref[pl.ds(i * bm, bm), :] = _soft_assign(z, mu)


def kernel(x, adj, W, b, mu):
    n, nfeat = x.shape
    nhid = W.shape[1]
    nclust = mu.shape[0]

    wbm = jnp.concatenate([W, b.reshape(1, nhid), mu], axis=0)

    bm = 400
    z, q = pl.pallas_call(
        functools.partial(_main_kernel, bm=bm, nfeat=nfeat, nclust=nclust),
        grid=(n // bm,),
        in_specs=[
            pl.BlockSpec((bm, n), lambda i: (i, 0)),
            pl.BlockSpec(memory_space=pl.ANY),
            pl.BlockSpec(memory_space=pl.ANY),
        ],
        out_specs=[
            pl.BlockSpec((n, nhid), lambda i: (0, 0)),
            pl.BlockSpec((n, nclust), lambda i: (0, 0)),
        ],
        out_shape=[
            jax.ShapeDtypeStruct((n, nhid), jnp.float32),
            jax.ShapeDtypeStruct((n, nclust), jnp.float32),
        ],
        scratch_shapes=[
            pltpu.VMEM((n, nfeat), jnp.float32),
            pltpu.VMEM((nfeat + 1 + nclust, nhid), jnp.float32),
            pltpu.VMEM((n, nhid), jnp.bfloat16),
            pltpu.SemaphoreType.DMA,
        ],
        compiler_params=pltpu.CompilerParams(
            dimension_semantics=("arbitrary",)),
    )(adj, x, wbm)
    return z, q
